# Initial kernel scaffold; baseline (speedup 1.0000x reference)
import jax
import jax.numpy as jnp
from jax.experimental import pallas as pl


def kernel(z, pos, batch, emb, W0, b0, W1, b1, W2, b2, rW0, rb0, rW1, rb1, rW2, rb2):
    raise NotImplementedError("write your pallas kernel here")


# speedup vs baseline: 9.1504x; 9.1504x over previous
import functools

import jax
import jax.numpy as jnp
from jax import lax
from jax.experimental import pallas as pl
from jax.experimental.pallas import tpu as pltpu
from jax.experimental.pallas import tpu_sc as plsc

N = 10000
NPAD = 10240
NB = NPAD // 128
BGRAPH = 64
KNN = 5

_NC = 2
_NS = 16
NW = _NC * _NS
BPW = NPAD // NW
CH = 80
NCH = BPW // CH

_F32 = jnp.float32
_I32 = jnp.int32
_INF = float("inf")
_PREC = jax.lax.Precision.DEFAULT


def _prep_body(emb_ref, w_ref, out_ref):
    out_ref[...] = jnp.dot(emb_ref[...], w_ref[...],
                           preferred_element_type=_F32, precision=_PREC)


def _prep_call(emb_pad, w0):
    return pl.pallas_call(
        _prep_body,
        out_shape=jax.ShapeDtypeStruct((104, 128), _F32),
    )(emb_pad, w0)


def _knn_body(lo_ref, nt_ref, posr_ref, batchr_ref, posT_ref, batchT_ref,
              out_ref):
    i = pl.program_id(0)
    lo = pl.multiple_of(lo_ref[i], 128)
    nt = nt_ref[i]
    pr = posr_ref[...]
    br = batchr_ref[0]
    sqr = jnp.sum(pr * pr, axis=1, keepdims=True)
    row_ids = lax.broadcasted_iota(_I32, (128, 1), 0) + i * 128

    init_d = jnp.full((128, 8), _INF, _F32)
    init_i = lax.broadcasted_iota(_I32, (128, 8), 1)
    wcol = lax.broadcasted_iota(_I32, (128, 136), 1)

    def tile(t, carry):
        bd, bi = carry
        c0 = pl.multiple_of(lo + t * 128, 128)
        pc = posT_ref[:, pl.ds(c0, 128)]
        bc = batchT_ref[pl.ds(0, 1), pl.ds(c0, 128)]
        sqc = jnp.sum(pc * pc, axis=0, keepdims=True)
        dot = lax.dot_general(pr, pc, (((1,), (0,)), ((), ())),
                              preferred_element_type=_F32, precision=_PREC)
        d2 = sqr + sqc - 2.0 * dot
        col_ids = lax.broadcasted_iota(_I32, (128, 128), 1) + c0
        valid = (bc == br) & (col_ids != row_ids)
        d2 = jnp.where(valid, d2, _INF)

        wd = jnp.concatenate([bd, d2], axis=1)
        wi = jnp.concatenate([bi, col_ids], axis=1)
        nd, ni = [], []
        for _ in range(KNN):
            m = jnp.min(wd, axis=1, keepdims=True)
            a = jnp.min(jnp.where(wd == m, wcol, 2 ** 30),
                        axis=1, keepdims=True)
            sel = wcol == a
            nd.append(m)
            ni.append(jnp.sum(jnp.where(sel, wi, 0), axis=1, keepdims=True))
            wd = jnp.where(sel, _INF, wd)
        pad_d = jnp.full((128, 3), _INF, _F32)
        pad_i = lax.broadcasted_iota(_I32, (128, 3), 1) + 5
        nbd = jnp.concatenate(nd + [pad_d], axis=1)
        nbi = jnp.concatenate(ni + [pad_i], axis=1)
        return nbd, nbi

    _, bi = lax.fori_loop(0, nt, tile, (init_d, init_i))
    out_ref[...] = bi


def _knn_call(lo, nt, pos8, batch3, posT8, batchT8):
    grid_spec = pltpu.PrefetchScalarGridSpec(
        num_scalar_prefetch=2,
        grid=(NB,),
        in_specs=[
            pl.BlockSpec((128, 8), lambda i, lo, nt: (i, 0)),
            pl.BlockSpec((1, 128, 1), lambda i, lo, nt: (i, 0, 0)),
            pl.BlockSpec((8, NPAD), lambda i, lo, nt: (0, 0)),
            pl.BlockSpec((8, NPAD), lambda i, lo, nt: (0, 0)),
        ],
        out_specs=pl.BlockSpec((128, 8), lambda i, lo, nt: (i, 0)),
    )
    return pl.pallas_call(
        _knn_body,
        grid_spec=grid_spec,
        out_shape=jax.ShapeDtypeStruct((NPAD, 8), _I32),
    )(lo, nt, pos8, batch3, posT8, batchT8)


@functools.cache
def _sc_mesh():
    return plsc.VectorSubcoreMesh(core_axis_name="c", subcore_axis_name="s")


def _lookup_kernel(table_hbm, idx_hbm, out_hbm, idx_v, rows_v, sem):
    wid = lax.axis_index("s") * _NC + lax.axis_index("c")
    base = wid * BPW
    pltpu.sync_copy(idx_hbm.at[wid], idx_v)
    for j in range(NCH):
        pltpu.async_copy(table_hbm.at[idx_v.at[j]],
                         rows_v.at[pl.ds(j * CH, CH)], sem).wait()
    pltpu.sync_copy(rows_v, out_hbm.at[pl.ds(base, BPW)])


def _lookup_call(table, idx3):
    k = functools.partial(
        pl.kernel,
        mesh=_sc_mesh(),
        out_type=jax.ShapeDtypeStruct((NPAD, 128), _F32),
        scratch_types=[
            pltpu.VMEM((NCH, CH), _I32),
            pltpu.VMEM((BPW, 128), _F32),
            pltpu.SemaphoreType.DMA,
        ],
    )(_lookup_kernel)
    return k(table, idx3)


def _agg_kernel(h_hbm, idx_hbm, out_hbm, idx_v, acc_v, buf_v, sem0, sem1):
    wid = lax.axis_index("s") * _NC + lax.axis_index("c")
    base = wid * BPW
    pltpu.sync_copy(idx_hbm.at[:, wid], idx_v)
    for j in range(NCH):
        pltpu.async_copy(h_hbm.at[idx_v.at[0, j]],
                         acc_v.at[pl.ds(j * CH, CH)], sem0).wait()
    for k in range(1, KNN):
        for j in range(NCH):
            pltpu.async_copy(h_hbm.at[idx_v.at[k, j]],
                             buf_v.at[pl.ds(j * CH, CH)], sem1).wait()

        def body(r, _):
            for c in range(0, 128, 16):
                acc_v[r, pl.ds(c, 16)] = (acc_v[r, pl.ds(c, 16)] +
                                          buf_v[r, pl.ds(c, 16)])
            return 0

        lax.fori_loop(0, BPW, body, 0, unroll=False)
    pltpu.sync_copy(acc_v, out_hbm.at[pl.ds(base, BPW)])


def _agg_call(h, idx4):
    k = functools.partial(
        pl.kernel,
        mesh=_sc_mesh(),
        out_type=jax.ShapeDtypeStruct((NPAD, 128), _F32),
        scratch_types=[
            pltpu.VMEM((KNN, NCH, CH), _I32),
            pltpu.VMEM((BPW, 128), _F32),
            pltpu.VMEM((BPW, 128), _F32),
            pltpu.SemaphoreType.DMA,
            pltpu.SemaphoreType.DMA,
        ],
    )(_agg_kernel)
    return k(h, idx4)


def _fused_body(acc_ref, h_ref, b_ref, w_ref, out_ref):
    x = jnp.maximum((acc_ref[...] + h_ref[...]) * (1.0 / 6.0) + b_ref[...],
                    0.0)
    out_ref[...] = jnp.dot(x, w_ref[...], preferred_element_type=_F32, precision=_PREC)


def _fused_call(acc, h, brow, wnext):
    return pl.pallas_call(
        _fused_body,
        grid=(NB,),
        in_specs=[
            pl.BlockSpec((128, 128), lambda i: (i, 0)),
            pl.BlockSpec((128, 128), lambda i: (i, 0)),
            pl.BlockSpec((1, 128), lambda i: (0, 0)),
            pl.BlockSpec((128, 128), lambda i: (0, 0)),
        ],
        out_specs=pl.BlockSpec((128, 128), lambda i: (i, 0)),
        out_shape=jax.ShapeDtypeStruct((NPAD, 128), _F32),
    )(acc, h, brow, wnext)


def _final_body(acc_ref, h_ref, b_ref, batch_ref, invc_ref,
                rw0_ref, rb0_ref, rw1_ref, rb1_ref, rw2_ref, rb2_ref,
                out_ref, pool_ref):
    i = pl.program_id(0)
    x = jnp.maximum((acc_ref[...] + h_ref[...]) * (1.0 / 6.0) + b_ref[...],
                    0.0)
    bcol = batch_ref[0]
    gids = lax.broadcasted_iota(_I32, (128, BGRAPH), 1)
    oh = (bcol == gids).astype(_F32)
    part = lax.dot_general(oh, x, (((0,), (0,)), ((), ())),
                           preferred_element_type=_F32, precision=_PREC)

    @pl.when(i == 0)
    def _():
        pool_ref[...] = jnp.zeros_like(pool_ref)

    pool_ref[...] += part

    @pl.when(i == NB - 1)
    def _():
        pooled = pool_ref[...] * invc_ref[...]
        hh = jnp.maximum(jnp.dot(pooled, rw0_ref[...],
                                 preferred_element_type=_F32, precision=_PREC) + rb0_ref[...],
                         0.0)
        gg = jnp.maximum(jnp.dot(hh, rw1_ref[...],
                                 preferred_element_type=_F32, precision=_PREC) + rb1_ref[...],
                         0.0)
        out_ref[...] = (jnp.dot(gg, rw2_ref[...],
                                preferred_element_type=_F32, precision=_PREC) + rb2_ref[...])


def _final_call(acc, h, brow, batch3, invc, rw0, rb0r, rw1, rb1r, rw2b, rb2r):
    return pl.pallas_call(
        _final_body,
        grid=(NB,),
        in_specs=[
            pl.BlockSpec((128, 128), lambda i: (i, 0)),
            pl.BlockSpec((128, 128), lambda i: (i, 0)),
            pl.BlockSpec((1, 128), lambda i: (0, 0)),
            pl.BlockSpec((1, 128, 1), lambda i: (i, 0, 0)),
            pl.BlockSpec((BGRAPH, 1), lambda i: (0, 0)),
            pl.BlockSpec((128, BGRAPH), lambda i: (0, 0)),
            pl.BlockSpec((1, BGRAPH), lambda i: (0, 0)),
            pl.BlockSpec((BGRAPH, 32), lambda i: (0, 0)),
            pl.BlockSpec((1, 32), lambda i: (0, 0)),
            pl.BlockSpec((32, 128), lambda i: (0, 0)),
            pl.BlockSpec((1, 128), lambda i: (0, 0)),
        ],
        out_specs=pl.BlockSpec((BGRAPH, 128), lambda i: (0, 0)),
        out_shape=jax.ShapeDtypeStruct((BGRAPH, 128), _F32),
        scratch_shapes=[pltpu.VMEM((BGRAPH, 128), _F32)],
    )(acc, h, brow, batch3, invc, rw0, rb0r, rw1, rb1r, rw2b, rb2r)


def kernel(z, pos, batch, emb, W0, b0, W1, b1, W2, b2,
           rW0, rb0, rW1, rb1, rW2, rb2):
    z = z.astype(_I32)
    batch = batch.astype(_I32)

    pos8 = jnp.zeros((NPAD, 8), _F32).at[:N, :3].set(pos)
    posT8 = pos8.T
    batch_pad = jnp.concatenate(
        [batch, jnp.full((NPAD - N,), BGRAPH, _I32)])
    batchT8 = jnp.broadcast_to(batch_pad, (8, NPAD))
    batch3 = batch_pad.reshape(NB, 128, 1)

    bounds = jnp.searchsorted(batch, jnp.arange(BGRAPH + 1)).astype(_I32)
    blk = jnp.arange(NB) * 128
    bfirst = jnp.clip(batch_pad[blk], 0, BGRAPH - 1)
    blast = jnp.clip(batch_pad[blk + 127], 0, BGRAPH - 1)
    lo = bounds[bfirst]
    hi = bounds[blast + 1]
    lo_al = (lo // 128) * 128
    nt = (hi - lo_al + 127) // 128

    cnts = (bounds[1:] - bounds[:-1]).astype(_F32)
    invc = (1.0 / jnp.maximum(cnts, 1.0)).reshape(BGRAPH, 1)

    z3 = jnp.concatenate([z, jnp.zeros((NPAD - N,), _I32)]).reshape(
        NW, NCH, CH)
    emb_pad = jnp.zeros((104, 128), _F32).at[:100].set(emb)

    b0r, b1r, b2r = b0.reshape(1, 128), b1.reshape(1, 128), b2.reshape(1, 128)
    rb0r, rb1r = rb0.reshape(1, BGRAPH), rb1.reshape(1, 32)
    rw2b = jnp.broadcast_to(rW2, (32, 128))
    rb2r = jnp.broadcast_to(rb2.reshape(1, 1), (1, 128))

    embw0 = _prep_call(emb_pad, W0)[:100]
    idx = _knn_call(lo_al, nt, pos8, batch3, posT8, batchT8)
    h0 = _lookup_call(embw0, z3)

    idx4 = idx.T[:KNN].reshape(KNN, NW, NCH, CH)

    acc0 = _agg_call(h0, idx4)
    h1 = _fused_call(acc0, h0, b0r, W1)
    acc1 = _agg_call(h1, idx4)
    h2 = _fused_call(acc1, h1, b1r, W2)
    acc2 = _agg_call(h2, idx4)
    res = _final_call(acc2, h2, b2r, batch3, invc,
                      rW0, rb0r, rW1, rb1r, rw2b, rb2r)
    return res[:, 0]


# speedup vs baseline: 9.5842x; 1.0474x over previous
import functools

import jax
import jax.numpy as jnp
from jax import lax
from jax.experimental import pallas as pl
from jax.experimental.pallas import tpu as pltpu
from jax.experimental.pallas import tpu_sc as plsc

N = 10000
NPAD = 10240
NB = NPAD // 128
BGRAPH = 64
KNN = 5

_NC = 2
_NS = 16
NW = _NC * _NS
BPW = NPAD // NW
CH = 80
NCH = BPW // CH

_F32 = jnp.float32
_I32 = jnp.int32
_INF = float("inf")
_PREC = jax.lax.Precision.DEFAULT


def _prep_body(emb_ref, w_ref, out_ref):
    out_ref[...] = jnp.dot(emb_ref[...], w_ref[...],
                           preferred_element_type=_F32, precision=_PREC)


def _prep_call(emb_pad, w0):
    return pl.pallas_call(
        _prep_body,
        out_shape=jax.ShapeDtypeStruct((104, 128), _F32),
    )(emb_pad, w0)


def _knn_body(lo_ref, nt_ref, posr_ref, batchr_ref, posT_ref, batchT_ref,
              out_ref):
    i = pl.program_id(0)
    lo = pl.multiple_of(lo_ref[i], 128)
    nt = nt_ref[i]
    pr = posr_ref[...]
    br = batchr_ref[0]
    sqr = jnp.sum(pr * pr, axis=1, keepdims=True)
    row_ids = lax.broadcasted_iota(_I32, (128, 1), 0) + i * 128

    init_d = jnp.full((128, 8), _INF, _F32)
    init_i = lax.broadcasted_iota(_I32, (128, 8), 1)
    wcol = lax.broadcasted_iota(_I32, (128, 136), 1)

    def tile(t, carry):
        bd, bi = carry
        c0 = pl.multiple_of(lo + t * 128, 128)
        pc = posT_ref[:, pl.ds(c0, 128)]
        bc = batchT_ref[pl.ds(0, 1), pl.ds(c0, 128)]
        sqc = jnp.sum(pc * pc, axis=0, keepdims=True)
        dot = lax.dot_general(pr, pc, (((1,), (0,)), ((), ())),
                              preferred_element_type=_F32, precision=_PREC)
        d2 = sqr + sqc - 2.0 * dot
        col_ids = lax.broadcasted_iota(_I32, (128, 128), 1) + c0
        valid = (bc == br) & (col_ids != row_ids)
        d2 = jnp.where(valid, d2, _INF)

        wd = jnp.concatenate([bd, d2], axis=1)
        wi = jnp.concatenate([bi, col_ids], axis=1)
        nd, ni = [], []
        for _ in range(KNN):
            m = jnp.min(wd, axis=1, keepdims=True)
            a = jnp.min(jnp.where(wd == m, wcol, 2 ** 30),
                        axis=1, keepdims=True)
            sel = wcol == a
            nd.append(m)
            ni.append(jnp.sum(jnp.where(sel, wi, 0), axis=1, keepdims=True))
            wd = jnp.where(sel, _INF, wd)
        pad_d = jnp.full((128, 3), _INF, _F32)
        pad_i = lax.broadcasted_iota(_I32, (128, 3), 1) + 5
        nbd = jnp.concatenate(nd + [pad_d], axis=1)
        nbi = jnp.concatenate(ni + [pad_i], axis=1)
        return nbd, nbi

    _, bi = lax.fori_loop(0, nt, tile, (init_d, init_i))
    out_ref[...] = bi


def _knn_call(lo, nt, pos8, batch3, posT8, batchT8):
    grid_spec = pltpu.PrefetchScalarGridSpec(
        num_scalar_prefetch=2,
        grid=(NB,),
        in_specs=[
            pl.BlockSpec((128, 8), lambda i, lo, nt: (i, 0)),
            pl.BlockSpec((1, 128, 1), lambda i, lo, nt: (i, 0, 0)),
            pl.BlockSpec((8, NPAD), lambda i, lo, nt: (0, 0)),
            pl.BlockSpec((8, NPAD), lambda i, lo, nt: (0, 0)),
        ],
        out_specs=pl.BlockSpec((128, 8), lambda i, lo, nt: (i, 0)),
    )
    return pl.pallas_call(
        _knn_body,
        grid_spec=grid_spec,
        out_shape=jax.ShapeDtypeStruct((NPAD, 8), _I32),
    )(lo, nt, pos8, batch3, posT8, batchT8)


@functools.cache
def _sc_mesh():
    return plsc.VectorSubcoreMesh(core_axis_name="c", subcore_axis_name="s")


def _lookup_kernel(table_hbm, idx_hbm, out_hbm, idx_v, rows_v, sem):
    wid = lax.axis_index("s") * _NC + lax.axis_index("c")
    base = wid * BPW
    pltpu.sync_copy(idx_hbm.at[wid], idx_v)
    for j in range(NCH):
        pltpu.async_copy(table_hbm.at[idx_v.at[j]],
                         rows_v.at[pl.ds(j * CH, CH)], sem).wait()
    pltpu.sync_copy(rows_v, out_hbm.at[pl.ds(base, BPW)])


def _lookup_call(table, idx3):
    k = functools.partial(
        pl.kernel,
        mesh=_sc_mesh(),
        out_type=jax.ShapeDtypeStruct((NPAD, 128), _F32),
        scratch_types=[
            pltpu.VMEM((NCH, CH), _I32),
            pltpu.VMEM((BPW, 128), _F32),
            pltpu.SemaphoreType.DMA,
        ],
    )(_lookup_kernel)
    return k(table, idx3)


def _agg_kernel(h_hbm, idx_hbm, out_hbm, idx_v, acc_v, sem0, sem1):
    wid = lax.axis_index("s") * _NC + lax.axis_index("c")
    base = wid * BPW
    pltpu.sync_copy(idx_hbm.at[:, wid], idx_v)
    copies = []
    for j in range(NCH):
        copies.append(pltpu.async_copy(h_hbm.at[idx_v.at[0, j]],
                                       acc_v.at[pl.ds(j * CH, CH)], sem0))
    for c in copies:
        c.wait()
    for k in range(1, KNN):
        copies = []
        for j in range(NCH):
            copies.append(pltpu.async_copy(h_hbm.at[idx_v.at[k, j]],
                                           acc_v.at[pl.ds(j * CH, CH)],
                                           sem1, add=True))
        for c in copies:
            c.wait()
    pltpu.sync_copy(acc_v, out_hbm.at[pl.ds(base, BPW)])


def _agg_call(h, idx4):
    k = functools.partial(
        pl.kernel,
        mesh=_sc_mesh(),
        out_type=jax.ShapeDtypeStruct((NPAD, 128), _F32),
        scratch_types=[
            pltpu.VMEM((KNN, NCH, CH), _I32),
            pltpu.VMEM((BPW, 128), _F32),
            pltpu.SemaphoreType.DMA,
            pltpu.SemaphoreType.DMA,
        ],
    )(_agg_kernel)
    return k(h, idx4)


def _fused_body(acc_ref, h_ref, b_ref, w_ref, out_ref):
    x = jnp.maximum((acc_ref[...] + h_ref[...]) * (1.0 / 6.0) + b_ref[...],
                    0.0)
    out_ref[...] = jnp.dot(x, w_ref[...], preferred_element_type=_F32, precision=_PREC)


def _fused_call(acc, h, brow, wnext):
    return pl.pallas_call(
        _fused_body,
        grid=(NB,),
        in_specs=[
            pl.BlockSpec((128, 128), lambda i: (i, 0)),
            pl.BlockSpec((128, 128), lambda i: (i, 0)),
            pl.BlockSpec((1, 128), lambda i: (0, 0)),
            pl.BlockSpec((128, 128), lambda i: (0, 0)),
        ],
        out_specs=pl.BlockSpec((128, 128), lambda i: (i, 0)),
        out_shape=jax.ShapeDtypeStruct((NPAD, 128), _F32),
    )(acc, h, brow, wnext)


def _final_body(acc_ref, h_ref, b_ref, batch_ref, invc_ref,
                rw0_ref, rb0_ref, rw1_ref, rb1_ref, rw2_ref, rb2_ref,
                out_ref, pool_ref):
    i = pl.program_id(0)
    x = jnp.maximum((acc_ref[...] + h_ref[...]) * (1.0 / 6.0) + b_ref[...],
                    0.0)
    bcol = batch_ref[0]
    gids = lax.broadcasted_iota(_I32, (128, BGRAPH), 1)
    oh = (bcol == gids).astype(_F32)
    part = lax.dot_general(oh, x, (((0,), (0,)), ((), ())),
                           preferred_element_type=_F32, precision=_PREC)

    @pl.when(i == 0)
    def _():
        pool_ref[...] = jnp.zeros_like(pool_ref)

    pool_ref[...] += part

    @pl.when(i == NB - 1)
    def _():
        pooled = pool_ref[...] * invc_ref[...]
        hh = jnp.maximum(jnp.dot(pooled, rw0_ref[...],
                                 preferred_element_type=_F32, precision=_PREC) + rb0_ref[...],
                         0.0)
        gg = jnp.maximum(jnp.dot(hh, rw1_ref[...],
                                 preferred_element_type=_F32, precision=_PREC) + rb1_ref[...],
                         0.0)
        out_ref[...] = (jnp.dot(gg, rw2_ref[...],
                                preferred_element_type=_F32, precision=_PREC) + rb2_ref[...])


def _final_call(acc, h, brow, batch3, invc, rw0, rb0r, rw1, rb1r, rw2b, rb2r):
    return pl.pallas_call(
        _final_body,
        grid=(NB,),
        in_specs=[
            pl.BlockSpec((128, 128), lambda i: (i, 0)),
            pl.BlockSpec((128, 128), lambda i: (i, 0)),
            pl.BlockSpec((1, 128), lambda i: (0, 0)),
            pl.BlockSpec((1, 128, 1), lambda i: (i, 0, 0)),
            pl.BlockSpec((BGRAPH, 1), lambda i: (0, 0)),
            pl.BlockSpec((128, BGRAPH), lambda i: (0, 0)),
            pl.BlockSpec((1, BGRAPH), lambda i: (0, 0)),
            pl.BlockSpec((BGRAPH, 32), lambda i: (0, 0)),
            pl.BlockSpec((1, 32), lambda i: (0, 0)),
            pl.BlockSpec((32, 128), lambda i: (0, 0)),
            pl.BlockSpec((1, 128), lambda i: (0, 0)),
        ],
        out_specs=pl.BlockSpec((BGRAPH, 128), lambda i: (0, 0)),
        out_shape=jax.ShapeDtypeStruct((BGRAPH, 128), _F32),
        scratch_shapes=[pltpu.VMEM((BGRAPH, 128), _F32)],
    )(acc, h, brow, batch3, invc, rw0, rb0r, rw1, rb1r, rw2b, rb2r)


def kernel(z, pos, batch, emb, W0, b0, W1, b1, W2, b2,
           rW0, rb0, rW1, rb1, rW2, rb2):
    z = z.astype(_I32)
    batch = batch.astype(_I32)

    pos8 = jnp.zeros((NPAD, 8), _F32).at[:N, :3].set(pos)
    posT8 = pos8.T
    batch_pad = jnp.concatenate(
        [batch, jnp.full((NPAD - N,), BGRAPH, _I32)])
    batchT8 = jnp.broadcast_to(batch_pad, (8, NPAD))
    batch3 = batch_pad.reshape(NB, 128, 1)

    bounds = jnp.searchsorted(batch, jnp.arange(BGRAPH + 1)).astype(_I32)
    blk = jnp.arange(NB) * 128
    bfirst = jnp.clip(batch_pad[blk], 0, BGRAPH - 1)
    blast = jnp.clip(batch_pad[blk + 127], 0, BGRAPH - 1)
    lo = bounds[bfirst]
    hi = bounds[blast + 1]
    lo_al = (lo // 128) * 128
    nt = (hi - lo_al + 127) // 128

    cnts = (bounds[1:] - bounds[:-1]).astype(_F32)
    invc = (1.0 / jnp.maximum(cnts, 1.0)).reshape(BGRAPH, 1)

    z3 = jnp.concatenate([z, jnp.zeros((NPAD - N,), _I32)]).reshape(
        NW, NCH, CH)
    emb_pad = jnp.zeros((104, 128), _F32).at[:100].set(emb)

    b0r, b1r, b2r = b0.reshape(1, 128), b1.reshape(1, 128), b2.reshape(1, 128)
    rb0r, rb1r = rb0.reshape(1, BGRAPH), rb1.reshape(1, 32)
    rw2b = jnp.broadcast_to(rW2, (32, 128))
    rb2r = jnp.broadcast_to(rb2.reshape(1, 1), (1, 128))

    embw0 = _prep_call(emb_pad, W0)[:100]
    idx = _knn_call(lo_al, nt, pos8, batch3, posT8, batchT8)
    h0 = _lookup_call(embw0, z3)

    idx4 = idx.T[:KNN].reshape(KNN, NW, NCH, CH)

    acc0 = _agg_call(h0, idx4)
    h1 = _fused_call(acc0, h0, b0r, W1)
    acc1 = _agg_call(h1, idx4)
    h2 = _fused_call(acc1, h1, b1r, W2)
    acc2 = _agg_call(h2, idx4)
    res = _final_call(acc2, h2, b2r, batch3, invc,
                      rW0, rb0r, rW1, rb1r, rw2b, rb2r)
    return res[:, 0]


# speedup vs baseline: 11.1005x; 1.1582x over previous
import functools

import jax
import jax.numpy as jnp
from jax import lax
from jax.experimental import pallas as pl
from jax.experimental.pallas import tpu as pltpu
from jax.experimental.pallas import tpu_sc as plsc

N = 10000
NPAD = 10240
NB = NPAD // 128
BGRAPH = 64
KNN = 5

_NC = 2
_NS = 16
NW = _NC * _NS
BPW = NPAD // NW
CH = 80
NCH = BPW // CH

_F32 = jnp.float32
_I32 = jnp.int32
_INF = float("inf")
_PREC = jax.lax.Precision.DEFAULT


def _prep_body(emb_ref, w_ref, out_ref):
    out_ref[...] = jnp.dot(emb_ref[...], w_ref[...],
                           preferred_element_type=_F32, precision=_PREC)


def _prep_call(emb_pad, w0):
    return pl.pallas_call(
        _prep_body,
        out_shape=jax.ShapeDtypeStruct((104, 128), _F32),
    )(emb_pad, w0)


def _knn_body(lo_ref, nt_ref, posr_ref, batchr_ref, posT_ref, batchT_ref,
              out_ref):
    i = pl.program_id(0)
    lo = pl.multiple_of(lo_ref[i], 128)
    nt = nt_ref[i]
    pr = posr_ref[...]
    br = batchr_ref[0]
    sqr = jnp.sum(pr * pr, axis=1, keepdims=True)
    row_ids = lax.broadcasted_iota(_I32, (128, 1), 0) + i * 128

    init_d = jnp.full((128, 8), _INF, _F32)
    col8f = lax.broadcasted_iota(_I32, (128, 8), 1).astype(_F32)
    wcolf = lax.broadcasted_iota(_I32, (128, 136), 1).astype(_F32)
    init_i = col8f

    def tile(t, carry):
        bd, bi = carry
        c0 = pl.multiple_of(lo + t * 128, 128)
        pc = posT_ref[:, pl.ds(c0, 128)]
        bc = batchT_ref[pl.ds(0, 1), pl.ds(c0, 128)]
        sqc = jnp.sum(pc * pc, axis=0, keepdims=True)
        dot = lax.dot_general(pr, pc, (((1,), (0,)), ((), ())),
                              preferred_element_type=_F32, precision=_PREC)
        d2 = sqr + sqc - 2.0 * dot
        col_ids = lax.broadcasted_iota(_I32, (128, 128), 1) + c0
        valid = (bc == br) & (col_ids != row_ids)
        d2 = jnp.where(valid, d2, _INF)
        c0f = (c0 - 8).astype(_F32)

        wd = jnp.concatenate([bd, d2], axis=1)
        nd, ni = [], []
        for _ in range(KNN):
            m = jnp.min(wd, axis=1, keepdims=True)
            a = jnp.min(jnp.where(wd == m, wcolf, 1e9),
                        axis=1, keepdims=True)
            ga = jnp.sum(jnp.where(col8f == a, bi, 0.0),
                         axis=1, keepdims=True)
            nd.append(m)
            ni.append(jnp.where(a < 8.0, ga, a + c0f))
            wd = jnp.where(wcolf == a, _INF, wd)
        pad_d = jnp.full((128, 3), _INF, _F32)
        pad_i = lax.broadcasted_iota(_I32, (128, 3), 1).astype(_F32) + 5.0
        nbd = jnp.concatenate(nd + [pad_d], axis=1)
        nbi = jnp.concatenate(ni + [pad_i], axis=1)
        return nbd, nbi

    _, bi = lax.fori_loop(0, nt, tile, (init_d, init_i))
    out_ref[...] = bi.astype(_I32)


def _knn_call(lo, nt, pos8, batch3, posT8, batchT8):
    grid_spec = pltpu.PrefetchScalarGridSpec(
        num_scalar_prefetch=2,
        grid=(NB,),
        in_specs=[
            pl.BlockSpec((128, 8), lambda i, lo, nt: (i, 0)),
            pl.BlockSpec((1, 128, 1), lambda i, lo, nt: (i, 0, 0)),
            pl.BlockSpec((8, NPAD), lambda i, lo, nt: (0, 0)),
            pl.BlockSpec((8, NPAD), lambda i, lo, nt: (0, 0)),
        ],
        out_specs=pl.BlockSpec((128, 8), lambda i, lo, nt: (i, 0)),
    )
    return pl.pallas_call(
        _knn_body,
        grid_spec=grid_spec,
        out_shape=jax.ShapeDtypeStruct((NPAD, 8), _I32),
    )(lo, nt, pos8, batch3, posT8, batchT8)


@functools.cache
def _sc_mesh():
    return plsc.VectorSubcoreMesh(core_axis_name="c", subcore_axis_name="s")


def _lookup_kernel(table_hbm, idx_hbm, out_hbm, idx_v, rows_v, sem):
    wid = lax.axis_index("s") * _NC + lax.axis_index("c")
    base = wid * BPW
    pltpu.sync_copy(idx_hbm.at[wid], idx_v)
    for j in range(NCH):
        pltpu.async_copy(table_hbm.at[idx_v.at[j]],
                         rows_v.at[pl.ds(j * CH, CH)], sem).wait()
    pltpu.sync_copy(rows_v, out_hbm.at[pl.ds(base, BPW)])


def _lookup_call(table, idx3):
    k = functools.partial(
        pl.kernel,
        mesh=_sc_mesh(),
        out_type=jax.ShapeDtypeStruct((NPAD, 128), _F32),
        scratch_types=[
            pltpu.VMEM((NCH, CH), _I32),
            pltpu.VMEM((BPW, 128), _F32),
            pltpu.SemaphoreType.DMA,
        ],
    )(_lookup_kernel)
    return k(table, idx3)


def _agg_kernel(h_hbm, idx_hbm, out_hbm, idx_v, acc_v, sem0, sem1):
    wid = lax.axis_index("s") * _NC + lax.axis_index("c")
    base = wid * BPW
    pltpu.sync_copy(idx_hbm.at[:, wid], idx_v)
    copies = []
    for j in range(NCH):
        copies.append(pltpu.async_copy(h_hbm.at[idx_v.at[0, j]],
                                       acc_v.at[pl.ds(j * CH, CH)], sem0))
    for c in copies:
        c.wait()
    for k in range(1, KNN):
        copies = []
        for j in range(NCH):
            copies.append(pltpu.async_copy(h_hbm.at[idx_v.at[k, j]],
                                           acc_v.at[pl.ds(j * CH, CH)],
                                           sem1, add=True))
        for c in copies:
            c.wait()
    pltpu.sync_copy(acc_v, out_hbm.at[pl.ds(base, BPW)])


def _agg_call(h, idx4):
    k = functools.partial(
        pl.kernel,
        mesh=_sc_mesh(),
        out_type=jax.ShapeDtypeStruct((NPAD, 128), _F32),
        scratch_types=[
            pltpu.VMEM((KNN, NCH, CH), _I32),
            pltpu.VMEM((BPW, 128), _F32),
            pltpu.SemaphoreType.DMA,
            pltpu.SemaphoreType.DMA,
        ],
    )(_agg_kernel)
    return k(h, idx4)


def _fused_body(acc_ref, h_ref, b_ref, w_ref, out_ref):
    x = jnp.maximum((acc_ref[...] + h_ref[...]) * (1.0 / 6.0) + b_ref[...],
                    0.0)
    out_ref[...] = jnp.dot(x, w_ref[...], preferred_element_type=_F32, precision=_PREC)


def _fused_call(acc, h, brow, wnext):
    return pl.pallas_call(
        _fused_body,
        grid=(NB,),
        in_specs=[
            pl.BlockSpec((128, 128), lambda i: (i, 0)),
            pl.BlockSpec((128, 128), lambda i: (i, 0)),
            pl.BlockSpec((1, 128), lambda i: (0, 0)),
            pl.BlockSpec((128, 128), lambda i: (0, 0)),
        ],
        out_specs=pl.BlockSpec((128, 128), lambda i: (i, 0)),
        out_shape=jax.ShapeDtypeStruct((NPAD, 128), _F32),
    )(acc, h, brow, wnext)


def _final_body(acc_ref, h_ref, b_ref, batch_ref, invc_ref,
                rw0_ref, rb0_ref, rw1_ref, rb1_ref, rw2_ref, rb2_ref,
                out_ref, pool_ref):
    i = pl.program_id(0)
    x = jnp.maximum((acc_ref[...] + h_ref[...]) * (1.0 / 6.0) + b_ref[...],
                    0.0)
    bcol = batch_ref[0]
    gids = lax.broadcasted_iota(_I32, (128, BGRAPH), 1)
    oh = (bcol == gids).astype(_F32)
    part = lax.dot_general(oh, x, (((0,), (0,)), ((), ())),
                           preferred_element_type=_F32, precision=_PREC)

    @pl.when(i == 0)
    def _():
        pool_ref[...] = jnp.zeros_like(pool_ref)

    pool_ref[...] += part

    @pl.when(i == NB - 1)
    def _():
        pooled = pool_ref[...] * invc_ref[...]
        hh = jnp.maximum(jnp.dot(pooled, rw0_ref[...],
                                 preferred_element_type=_F32, precision=_PREC) + rb0_ref[...],
                         0.0)
        gg = jnp.maximum(jnp.dot(hh, rw1_ref[...],
                                 preferred_element_type=_F32, precision=_PREC) + rb1_ref[...],
                         0.0)
        out_ref[...] = (jnp.dot(gg, rw2_ref[...],
                                preferred_element_type=_F32, precision=_PREC) + rb2_ref[...])


def _final_call(acc, h, brow, batch3, invc, rw0, rb0r, rw1, rb1r, rw2b, rb2r):
    return pl.pallas_call(
        _final_body,
        grid=(NB,),
        in_specs=[
            pl.BlockSpec((128, 128), lambda i: (i, 0)),
            pl.BlockSpec((128, 128), lambda i: (i, 0)),
            pl.BlockSpec((1, 128), lambda i: (0, 0)),
            pl.BlockSpec((1, 128, 1), lambda i: (i, 0, 0)),
            pl.BlockSpec((BGRAPH, 1), lambda i: (0, 0)),
            pl.BlockSpec((128, BGRAPH), lambda i: (0, 0)),
            pl.BlockSpec((1, BGRAPH), lambda i: (0, 0)),
            pl.BlockSpec((BGRAPH, 32), lambda i: (0, 0)),
            pl.BlockSpec((1, 32), lambda i: (0, 0)),
            pl.BlockSpec((32, 128), lambda i: (0, 0)),
            pl.BlockSpec((1, 128), lambda i: (0, 0)),
        ],
        out_specs=pl.BlockSpec((BGRAPH, 128), lambda i: (0, 0)),
        out_shape=jax.ShapeDtypeStruct((BGRAPH, 128), _F32),
        scratch_shapes=[pltpu.VMEM((BGRAPH, 128), _F32)],
    )(acc, h, brow, batch3, invc, rw0, rb0r, rw1, rb1r, rw2b, rb2r)


def kernel(z, pos, batch, emb, W0, b0, W1, b1, W2, b2,
           rW0, rb0, rW1, rb1, rW2, rb2):
    z = z.astype(_I32)
    batch = batch.astype(_I32)

    pos8 = jnp.zeros((NPAD, 8), _F32).at[:N, :3].set(pos)
    posT8 = pos8.T
    batch_pad = jnp.concatenate(
        [batch, jnp.full((NPAD - N,), BGRAPH, _I32)])
    batchT8 = jnp.broadcast_to(batch_pad, (8, NPAD))
    batch3 = batch_pad.reshape(NB, 128, 1)

    bounds = jnp.searchsorted(batch, jnp.arange(BGRAPH + 1)).astype(_I32)
    blk = jnp.arange(NB) * 128
    bfirst = jnp.clip(batch_pad[blk], 0, BGRAPH - 1)
    blast = jnp.clip(batch_pad[blk + 127], 0, BGRAPH - 1)
    lo = bounds[bfirst]
    hi = bounds[blast + 1]
    lo_al = (lo // 128) * 128
    nt = (hi - lo_al + 127) // 128

    cnts = (bounds[1:] - bounds[:-1]).astype(_F32)
    invc = (1.0 / jnp.maximum(cnts, 1.0)).reshape(BGRAPH, 1)

    z3 = jnp.concatenate([z, jnp.zeros((NPAD - N,), _I32)]).reshape(
        NW, NCH, CH)
    emb_pad = jnp.zeros((104, 128), _F32).at[:100].set(emb)

    b0r, b1r, b2r = b0.reshape(1, 128), b1.reshape(1, 128), b2.reshape(1, 128)
    rb0r, rb1r = rb0.reshape(1, BGRAPH), rb1.reshape(1, 32)
    rw2b = jnp.broadcast_to(rW2, (32, 128))
    rb2r = jnp.broadcast_to(rb2.reshape(1, 1), (1, 128))

    embw0 = _prep_call(emb_pad, W0)[:100]
    idx = _knn_call(lo_al, nt, pos8, batch3, posT8, batchT8)
    h0 = _lookup_call(embw0, z3)

    idx4 = idx.T[:KNN].reshape(KNN, NW, NCH, CH)

    acc0 = _agg_call(h0, idx4)
    h1 = _fused_call(acc0, h0, b0r, W1)
    acc1 = _agg_call(h1, idx4)
    h2 = _fused_call(acc1, h1, b1r, W2)
    acc2 = _agg_call(h2, idx4)
    res = _final_call(acc2, h2, b2r, batch3, invc,
                      rW0, rb0r, rW1, rb1r, rw2b, rb2r)
    return res[:, 0]


# speedup vs baseline: 12.0277x; 1.0835x over previous
import functools

import jax
import jax.numpy as jnp
from jax import lax
from jax.experimental import pallas as pl
from jax.experimental.pallas import tpu as pltpu
from jax.experimental.pallas import tpu_sc as plsc

N = 10000
NPAD = 10240
NB = NPAD // 128
BGRAPH = 64
KNN = 5

_NC = 2
_NS = 16
NW = _NC * _NS
BPW = NPAD // NW
CH = 80
NCH = BPW // CH

_F32 = jnp.float32
_I32 = jnp.int32
_INF = float("inf")
_PREC = jax.lax.Precision.DEFAULT


def _prep_body(emb_ref, w_ref, out_ref):
    out_ref[...] = jnp.dot(emb_ref[...], w_ref[...],
                           preferred_element_type=_F32, precision=_PREC)


def _prep_call(emb_pad, w0):
    return pl.pallas_call(
        _prep_body,
        out_shape=jax.ShapeDtypeStruct((104, 128), _F32),
    )(emb_pad, w0)


def _knn_body(lo_ref, nt_ref, posr_ref, batchr_ref, posT_ref, batchT_ref,
              out_ref):
    i = pl.program_id(0)
    lo = pl.multiple_of(lo_ref[i], 128)
    nt = nt_ref[i]
    pr = posr_ref[...]
    br = batchr_ref[0]
    sqr = jnp.sum(pr * pr, axis=1, keepdims=True)
    row_ids = lax.broadcasted_iota(_I32, (128, 1), 0) + i * 128

    init_d = jnp.full((128, 8), _INF, _F32)
    col8f = lax.broadcasted_iota(_I32, (128, 8), 1).astype(_F32)
    wcolf = lax.broadcasted_iota(_I32, (128, 136), 1).astype(_F32)
    init_i = col8f

    def tile(t, carry):
        bd, bi = carry
        c0 = pl.multiple_of(lo + t * 128, 128)
        pc = posT_ref[:, pl.ds(c0, 128)]
        bc = batchT_ref[pl.ds(0, 1), pl.ds(c0, 128)]
        sqc = jnp.sum(pc * pc, axis=0, keepdims=True)
        dot = lax.dot_general(pr, pc, (((1,), (0,)), ((), ())),
                              preferred_element_type=_F32, precision=_PREC)
        d2 = sqr + sqc - 2.0 * dot
        col_ids = lax.broadcasted_iota(_I32, (128, 128), 1) + c0
        valid = (bc == br) & (col_ids != row_ids)
        d2 = jnp.where(valid, d2, _INF)
        c0f = (c0 - 8).astype(_F32)

        wd = jnp.concatenate([bd, d2], axis=1)
        nd, ni = [], []
        for _ in range(KNN):
            m = jnp.min(wd, axis=1, keepdims=True)
            a = jnp.min(jnp.where(wd == m, wcolf, 1e9),
                        axis=1, keepdims=True)
            ga = jnp.sum(jnp.where(col8f == a, bi, 0.0),
                         axis=1, keepdims=True)
            nd.append(m)
            ni.append(jnp.where(a < 8.0, ga, a + c0f))
            wd = jnp.where(wcolf == a, _INF, wd)
        pad_d = jnp.full((128, 3), _INF, _F32)
        pad_i = lax.broadcasted_iota(_I32, (128, 3), 1).astype(_F32) + 5.0
        nbd = jnp.concatenate(nd + [pad_d], axis=1)
        nbi = jnp.concatenate(ni + [pad_i], axis=1)
        return nbd, nbi

    _, bi = lax.fori_loop(0, nt, tile, (init_d, init_i))
    out_ref[...] = bi.astype(_I32)


def _knn_call(lo, nt, pos8, batch3, posT8, batchT8):
    grid_spec = pltpu.PrefetchScalarGridSpec(
        num_scalar_prefetch=2,
        grid=(NB,),
        in_specs=[
            pl.BlockSpec((128, 8), lambda i, lo, nt: (i, 0)),
            pl.BlockSpec((1, 128, 1), lambda i, lo, nt: (i, 0, 0)),
            pl.BlockSpec((8, NPAD), lambda i, lo, nt: (0, 0)),
            pl.BlockSpec((8, NPAD), lambda i, lo, nt: (0, 0)),
        ],
        out_specs=pl.BlockSpec((128, 8), lambda i, lo, nt: (i, 0)),
    )
    return pl.pallas_call(
        _knn_body,
        grid_spec=grid_spec,
        out_shape=jax.ShapeDtypeStruct((NPAD, 8), _I32),
    )(lo, nt, pos8, batch3, posT8, batchT8)


@functools.cache
def _sc_mesh():
    return plsc.VectorSubcoreMesh(core_axis_name="c", subcore_axis_name="s")


def _lookup_kernel(table_hbm, idx_hbm, out_hbm, idx_v, rows_v, sem):
    wid = lax.axis_index("s") * _NC + lax.axis_index("c")
    base = wid * BPW
    pltpu.sync_copy(idx_hbm.at[wid], idx_v)
    for j in range(NCH):
        pltpu.async_copy(table_hbm.at[idx_v.at[j]],
                         rows_v.at[pl.ds(j * CH, CH)], sem).wait()
    pltpu.sync_copy(rows_v, out_hbm.at[pl.ds(base, BPW)])


def _lookup_call(table, idx3):
    k = functools.partial(
        pl.kernel,
        mesh=_sc_mesh(),
        out_type=jax.ShapeDtypeStruct((NPAD, 128), _F32),
        scratch_types=[
            pltpu.VMEM((NCH, CH), _I32),
            pltpu.VMEM((BPW, 128), _F32),
            pltpu.SemaphoreType.DMA,
        ],
    )(_lookup_kernel)
    return k(table, idx3)


def _agg_kernel(h_hbm, idx_hbm, out_hbm, idx_v, acc_v, sem0, sem1):
    wid = lax.axis_index("s") * _NC + lax.axis_index("c")
    base = wid * BPW
    pltpu.sync_copy(idx_hbm.at[:, wid], idx_v)
    copies = []
    for j in range(NCH):
        copies.append(pltpu.async_copy(h_hbm.at[idx_v.at[0, j]],
                                       acc_v.at[pl.ds(j * CH, CH)], sem0))
    for c in copies:
        c.wait()
    copies = []
    for k in range(1, KNN):
        for j in range(NCH):
            copies.append(pltpu.async_copy(h_hbm.at[idx_v.at[k, j]],
                                           acc_v.at[pl.ds(j * CH, CH)],
                                           sem1, add=True))
    for c in copies:
        c.wait()
    pltpu.sync_copy(acc_v, out_hbm.at[pl.ds(base, BPW)])


def _agg_call(h, idx4):
    k = functools.partial(
        pl.kernel,
        mesh=_sc_mesh(),
        out_type=jax.ShapeDtypeStruct((NPAD, 128), _F32),
        scratch_types=[
            pltpu.VMEM((KNN, NCH, CH), _I32),
            pltpu.VMEM((BPW, 128), _F32),
            pltpu.SemaphoreType.DMA,
            pltpu.SemaphoreType.DMA,
        ],
    )(_agg_kernel)
    return k(h, idx4)


def _fused_body(acc_ref, h_ref, b_ref, w_ref, out_ref):
    x = jnp.maximum((acc_ref[...] + h_ref[...]) * (1.0 / 6.0) + b_ref[...],
                    0.0)
    out_ref[...] = jnp.dot(x, w_ref[...], preferred_element_type=_F32, precision=_PREC)


def _fused_call(acc, h, brow, wnext):
    return pl.pallas_call(
        _fused_body,
        grid=(NB,),
        in_specs=[
            pl.BlockSpec((128, 128), lambda i: (i, 0)),
            pl.BlockSpec((128, 128), lambda i: (i, 0)),
            pl.BlockSpec((1, 128), lambda i: (0, 0)),
            pl.BlockSpec((128, 128), lambda i: (0, 0)),
        ],
        out_specs=pl.BlockSpec((128, 128), lambda i: (i, 0)),
        out_shape=jax.ShapeDtypeStruct((NPAD, 128), _F32),
    )(acc, h, brow, wnext)


def _final_body(acc_ref, h_ref, b_ref, batch_ref, invc_ref,
                rw0_ref, rb0_ref, rw1_ref, rb1_ref, rw2_ref, rb2_ref,
                out_ref, pool_ref):
    i = pl.program_id(0)
    x = jnp.maximum((acc_ref[...] + h_ref[...]) * (1.0 / 6.0) + b_ref[...],
                    0.0)
    bcol = batch_ref[0]
    gids = lax.broadcasted_iota(_I32, (128, BGRAPH), 1)
    oh = (bcol == gids).astype(_F32)
    part = lax.dot_general(oh, x, (((0,), (0,)), ((), ())),
                           preferred_element_type=_F32, precision=_PREC)

    @pl.when(i == 0)
    def _():
        pool_ref[...] = jnp.zeros_like(pool_ref)

    pool_ref[...] += part

    @pl.when(i == NB - 1)
    def _():
        pooled = pool_ref[...] * invc_ref[...]
        hh = jnp.maximum(jnp.dot(pooled, rw0_ref[...],
                                 preferred_element_type=_F32, precision=_PREC) + rb0_ref[...],
                         0.0)
        gg = jnp.maximum(jnp.dot(hh, rw1_ref[...],
                                 preferred_element_type=_F32, precision=_PREC) + rb1_ref[...],
                         0.0)
        out_ref[...] = (jnp.dot(gg, rw2_ref[...],
                                preferred_element_type=_F32, precision=_PREC) + rb2_ref[...])


def _final_call(acc, h, brow, batch3, invc, rw0, rb0r, rw1, rb1r, rw2b, rb2r):
    return pl.pallas_call(
        _final_body,
        grid=(NB,),
        in_specs=[
            pl.BlockSpec((128, 128), lambda i: (i, 0)),
            pl.BlockSpec((128, 128), lambda i: (i, 0)),
            pl.BlockSpec((1, 128), lambda i: (0, 0)),
            pl.BlockSpec((1, 128, 1), lambda i: (i, 0, 0)),
            pl.BlockSpec((BGRAPH, 1), lambda i: (0, 0)),
            pl.BlockSpec((128, BGRAPH), lambda i: (0, 0)),
            pl.BlockSpec((1, BGRAPH), lambda i: (0, 0)),
            pl.BlockSpec((BGRAPH, 32), lambda i: (0, 0)),
            pl.BlockSpec((1, 32), lambda i: (0, 0)),
            pl.BlockSpec((32, 128), lambda i: (0, 0)),
            pl.BlockSpec((1, 128), lambda i: (0, 0)),
        ],
        out_specs=pl.BlockSpec((BGRAPH, 128), lambda i: (0, 0)),
        out_shape=jax.ShapeDtypeStruct((BGRAPH, 128), _F32),
        scratch_shapes=[pltpu.VMEM((BGRAPH, 128), _F32)],
    )(acc, h, brow, batch3, invc, rw0, rb0r, rw1, rb1r, rw2b, rb2r)


def kernel(z, pos, batch, emb, W0, b0, W1, b1, W2, b2,
           rW0, rb0, rW1, rb1, rW2, rb2):
    z = z.astype(_I32)
    batch = batch.astype(_I32)

    pos8 = jnp.zeros((NPAD, 8), _F32).at[:N, :3].set(pos)
    posT8 = pos8.T
    batch_pad = jnp.concatenate(
        [batch, jnp.full((NPAD - N,), BGRAPH, _I32)])
    batchT8 = jnp.broadcast_to(batch_pad, (8, NPAD))
    batch3 = batch_pad.reshape(NB, 128, 1)

    bounds = jnp.searchsorted(batch, jnp.arange(BGRAPH + 1)).astype(_I32)
    blk = jnp.arange(NB) * 128
    bfirst = jnp.clip(batch_pad[blk], 0, BGRAPH - 1)
    blast = jnp.clip(batch_pad[blk + 127], 0, BGRAPH - 1)
    lo = bounds[bfirst]
    hi = bounds[blast + 1]
    lo_al = (lo // 128) * 128
    nt = (hi - lo_al + 127) // 128

    cnts = (bounds[1:] - bounds[:-1]).astype(_F32)
    invc = (1.0 / jnp.maximum(cnts, 1.0)).reshape(BGRAPH, 1)

    z3 = jnp.concatenate([z, jnp.zeros((NPAD - N,), _I32)]).reshape(
        NW, NCH, CH)
    emb_pad = jnp.zeros((104, 128), _F32).at[:100].set(emb)

    b0r, b1r, b2r = b0.reshape(1, 128), b1.reshape(1, 128), b2.reshape(1, 128)
    rb0r, rb1r = rb0.reshape(1, BGRAPH), rb1.reshape(1, 32)
    rw2b = jnp.broadcast_to(rW2, (32, 128))
    rb2r = jnp.broadcast_to(rb2.reshape(1, 1), (1, 128))

    embw0 = _prep_call(emb_pad, W0)[:100]
    idx = _knn_call(lo_al, nt, pos8, batch3, posT8, batchT8)
    h0 = _lookup_call(embw0, z3)

    idx4 = idx.T[:KNN].reshape(KNN, NW, NCH, CH)

    acc0 = _agg_call(h0, idx4)
    h1 = _fused_call(acc0, h0, b0r, W1)
    acc1 = _agg_call(h1, idx4)
    h2 = _fused_call(acc1, h1, b1r, W2)
    acc2 = _agg_call(h2, idx4)
    res = _final_call(acc2, h2, b2r, batch3, invc,
                      rW0, rb0r, rW1, rb1r, rw2b, rb2r)
    return res[:, 0]


# speedup vs baseline: 14.3196x; 1.1905x over previous
import functools

import jax
import jax.numpy as jnp
from jax import lax
from jax.experimental import pallas as pl
from jax.experimental.pallas import tpu as pltpu
from jax.experimental.pallas import tpu_sc as plsc

N = 10000
NPAD = 10240
NB = NPAD // 128
BGRAPH = 64
KNN = 5

_NC = 2
_NS = 16
NW = _NC * _NS
BPW = NPAD // NW
CH = 80
NCH = BPW // CH

_F32 = jnp.float32
_I32 = jnp.int32
_INF = float("inf")
_PREC = jax.lax.Precision.DEFAULT


def _prep_body(emb_ref, w_ref, out_ref):
    out_ref[...] = jnp.dot(emb_ref[...], w_ref[...],
                           preferred_element_type=_F32, precision=_PREC)


def _prep_call(emb_pad, w0):
    return pl.pallas_call(
        _prep_body,
        out_shape=jax.ShapeDtypeStruct((104, 128), _F32),
    )(emb_pad, w0)


def _knn_body(lo_ref, nt_ref, posr_ref, batchr_ref, posT_ref, batchT_ref,
              out_ref):
    i = pl.program_id(0)
    lo = pl.multiple_of(lo_ref[i], 128)
    nt = nt_ref[i]
    pr = posr_ref[...]
    br = batchr_ref[0]
    sqr = jnp.sum(pr * pr, axis=1, keepdims=True)
    row_ids = lax.broadcasted_iota(_I32, (128, 1), 0) + i * 128

    init_d = jnp.full((128, 8), _INF, _F32)
    col8f = lax.broadcasted_iota(_I32, (128, 8), 1).astype(_F32)
    wcolf = lax.broadcasted_iota(_I32, (128, 264), 1).astype(_F32)
    init_i = col8f

    def tile(t, carry):
        bd, bi = carry
        c0 = pl.multiple_of(lo + t * 256, 128)
        pc = posT_ref[:, pl.ds(c0, 256)]
        bc = batchT_ref[pl.ds(0, 1), pl.ds(c0, 256)]
        sqc = jnp.sum(pc * pc, axis=0, keepdims=True)
        dot = lax.dot_general(pr, pc, (((1,), (0,)), ((), ())),
                              preferred_element_type=_F32, precision=_PREC)
        d2 = sqr + sqc - 2.0 * dot
        col_ids = lax.broadcasted_iota(_I32, (128, 256), 1) + c0
        valid = (bc == br) & (col_ids != row_ids)
        d2 = jnp.where(valid, d2, _INF)
        c0f = (c0 - 8).astype(_F32)

        wd = jnp.concatenate([bd, d2], axis=1)
        nd, ni = [], []
        for _ in range(KNN):
            m = jnp.min(wd, axis=1, keepdims=True)
            a = jnp.min(jnp.where(wd == m, wcolf, 1e9),
                        axis=1, keepdims=True)
            ga = jnp.sum(jnp.where(col8f == a, bi, 0.0),
                         axis=1, keepdims=True)
            nd.append(m)
            ni.append(jnp.where(a < 8.0, ga, a + c0f))
            wd = jnp.where(wcolf == a, _INF, wd)
        pad_d = jnp.full((128, 3), _INF, _F32)
        pad_i = lax.broadcasted_iota(_I32, (128, 3), 1).astype(_F32) + 5.0
        nbd = jnp.concatenate(nd + [pad_d], axis=1)
        nbi = jnp.concatenate(ni + [pad_i], axis=1)
        return nbd, nbi

    _, bi = lax.fori_loop(0, nt, tile, (init_d, init_i))
    out_ref[...] = bi.astype(_I32)


def _knn_call(lo, nt, pos8, batch3, posT8, batchT8):
    grid_spec = pltpu.PrefetchScalarGridSpec(
        num_scalar_prefetch=2,
        grid=(NB,),
        in_specs=[
            pl.BlockSpec((128, 8), lambda i, lo, nt: (i, 0)),
            pl.BlockSpec((1, 128, 1), lambda i, lo, nt: (i, 0, 0)),
            pl.BlockSpec((8, NPAD), lambda i, lo, nt: (0, 0)),
            pl.BlockSpec((8, NPAD), lambda i, lo, nt: (0, 0)),
        ],
        out_specs=pl.BlockSpec((128, 8), lambda i, lo, nt: (i, 0)),
    )
    return pl.pallas_call(
        _knn_body,
        grid_spec=grid_spec,
        out_shape=jax.ShapeDtypeStruct((NPAD, 8), _I32),
    )(lo, nt, pos8, batch3, posT8, batchT8)


@functools.cache
def _sc_mesh():
    return plsc.VectorSubcoreMesh(core_axis_name="c", subcore_axis_name="s")


def _lookup_kernel(table_hbm, idx_hbm, out_hbm, idx_v, rows_v, sem):
    wid = lax.axis_index("s") * _NC + lax.axis_index("c")
    base = wid * BPW
    pltpu.sync_copy(idx_hbm.at[wid], idx_v)
    for j in range(NCH):
        pltpu.async_copy(table_hbm.at[idx_v.at[j]],
                         rows_v.at[pl.ds(j * CH, CH)], sem).wait()
    pltpu.sync_copy(rows_v, out_hbm.at[pl.ds(base, BPW)])


def _lookup_call(table, idx3):
    k = functools.partial(
        pl.kernel,
        mesh=_sc_mesh(),
        out_type=jax.ShapeDtypeStruct((NPAD, 128), _F32),
        scratch_types=[
            pltpu.VMEM((NCH, CH), _I32),
            pltpu.VMEM((BPW, 128), _F32),
            pltpu.SemaphoreType.DMA,
        ],
    )(_lookup_kernel)
    return k(table, idx3)


def _agg_kernel(h_hbm, idx_hbm, out_hbm, idx_v, acc_v, sem0, sem1):
    wid = lax.axis_index("s") * _NC + lax.axis_index("c")
    base = wid * BPW
    pltpu.sync_copy(idx_hbm.at[:, wid], idx_v)
    copies = []
    for j in range(NCH):
        copies.append(pltpu.async_copy(h_hbm.at[idx_v.at[0, j]],
                                       acc_v.at[pl.ds(j * CH, CH)], sem0))
    for c in copies:
        c.wait()
    copies = []
    for k in range(1, KNN):
        for j in range(NCH):
            copies.append(pltpu.async_copy(h_hbm.at[idx_v.at[k, j]],
                                           acc_v.at[pl.ds(j * CH, CH)],
                                           sem1, add=True))
    for c in copies:
        c.wait()
    pltpu.sync_copy(acc_v, out_hbm.at[pl.ds(base, BPW)])


def _agg_call(h, idx4):
    k = functools.partial(
        pl.kernel,
        mesh=_sc_mesh(),
        out_type=jax.ShapeDtypeStruct((NPAD, 128), _F32),
        scratch_types=[
            pltpu.VMEM((KNN, NCH, CH), _I32),
            pltpu.VMEM((BPW, 128), _F32),
            pltpu.SemaphoreType.DMA,
            pltpu.SemaphoreType.DMA,
        ],
    )(_agg_kernel)
    return k(h, idx4)


def _fused_body(acc_ref, h_ref, b_ref, w_ref, out_ref):
    x = jnp.maximum((acc_ref[...] + h_ref[...]) * (1.0 / 6.0) + b_ref[...],
                    0.0)
    out_ref[...] = jnp.dot(x, w_ref[...], preferred_element_type=_F32, precision=_PREC)


def _fused_call(acc, h, brow, wnext):
    return pl.pallas_call(
        _fused_body,
        grid=(NB,),
        in_specs=[
            pl.BlockSpec((128, 128), lambda i: (i, 0)),
            pl.BlockSpec((128, 128), lambda i: (i, 0)),
            pl.BlockSpec((1, 128), lambda i: (0, 0)),
            pl.BlockSpec((128, 128), lambda i: (0, 0)),
        ],
        out_specs=pl.BlockSpec((128, 128), lambda i: (i, 0)),
        out_shape=jax.ShapeDtypeStruct((NPAD, 128), _F32),
    )(acc, h, brow, wnext)


def _final_body(acc_ref, h_ref, b_ref, batch_ref, invc_ref,
                rw0_ref, rb0_ref, rw1_ref, rb1_ref, rw2_ref, rb2_ref,
                out_ref, pool_ref):
    i = pl.program_id(0)
    x = jnp.maximum((acc_ref[...] + h_ref[...]) * (1.0 / 6.0) + b_ref[...],
                    0.0)
    bcol = batch_ref[0]
    gids = lax.broadcasted_iota(_I32, (128, BGRAPH), 1)
    oh = (bcol == gids).astype(_F32)
    part = lax.dot_general(oh, x, (((0,), (0,)), ((), ())),
                           preferred_element_type=_F32, precision=_PREC)

    @pl.when(i == 0)
    def _():
        pool_ref[...] = jnp.zeros_like(pool_ref)

    pool_ref[...] += part

    @pl.when(i == NB - 1)
    def _():
        pooled = pool_ref[...] * invc_ref[...]
        hh = jnp.maximum(jnp.dot(pooled, rw0_ref[...],
                                 preferred_element_type=_F32, precision=_PREC) + rb0_ref[...],
                         0.0)
        gg = jnp.maximum(jnp.dot(hh, rw1_ref[...],
                                 preferred_element_type=_F32, precision=_PREC) + rb1_ref[...],
                         0.0)
        out_ref[...] = (jnp.dot(gg, rw2_ref[...],
                                preferred_element_type=_F32, precision=_PREC) + rb2_ref[...])


def _final_call(acc, h, brow, batch3, invc, rw0, rb0r, rw1, rb1r, rw2b, rb2r):
    return pl.pallas_call(
        _final_body,
        grid=(NB,),
        in_specs=[
            pl.BlockSpec((128, 128), lambda i: (i, 0)),
            pl.BlockSpec((128, 128), lambda i: (i, 0)),
            pl.BlockSpec((1, 128), lambda i: (0, 0)),
            pl.BlockSpec((1, 128, 1), lambda i: (i, 0, 0)),
            pl.BlockSpec((BGRAPH, 1), lambda i: (0, 0)),
            pl.BlockSpec((128, BGRAPH), lambda i: (0, 0)),
            pl.BlockSpec((1, BGRAPH), lambda i: (0, 0)),
            pl.BlockSpec((BGRAPH, 32), lambda i: (0, 0)),
            pl.BlockSpec((1, 32), lambda i: (0, 0)),
            pl.BlockSpec((32, 128), lambda i: (0, 0)),
            pl.BlockSpec((1, 128), lambda i: (0, 0)),
        ],
        out_specs=pl.BlockSpec((BGRAPH, 128), lambda i: (0, 0)),
        out_shape=jax.ShapeDtypeStruct((BGRAPH, 128), _F32),
        scratch_shapes=[pltpu.VMEM((BGRAPH, 128), _F32)],
    )(acc, h, brow, batch3, invc, rw0, rb0r, rw1, rb1r, rw2b, rb2r)


def kernel(z, pos, batch, emb, W0, b0, W1, b1, W2, b2,
           rW0, rb0, rW1, rb1, rW2, rb2):
    z = z.astype(_I32)
    batch = batch.astype(_I32)

    pos8 = jnp.zeros((NPAD, 8), _F32).at[:N, :3].set(pos)
    posT8 = pos8.T
    batch_pad = jnp.concatenate(
        [batch, jnp.full((NPAD - N,), BGRAPH, _I32)])
    batchT8 = jnp.broadcast_to(batch_pad, (8, NPAD))
    batch3 = batch_pad.reshape(NB, 128, 1)

    bounds = jnp.searchsorted(batch, jnp.arange(BGRAPH + 1)).astype(_I32)
    blk = jnp.arange(NB) * 128
    bfirst = jnp.clip(batch_pad[blk], 0, BGRAPH - 1)
    blast = jnp.clip(batch_pad[blk + 127], 0, BGRAPH - 1)
    lo = bounds[bfirst]
    hi = bounds[blast + 1]
    lo_al = (lo // 128) * 128
    nt = (hi - lo_al + 255) // 256

    cnts = (bounds[1:] - bounds[:-1]).astype(_F32)
    invc = (1.0 / jnp.maximum(cnts, 1.0)).reshape(BGRAPH, 1)

    z3 = jnp.concatenate([z, jnp.zeros((NPAD - N,), _I32)]).reshape(
        NW, NCH, CH)
    emb_pad = jnp.zeros((104, 128), _F32).at[:100].set(emb)

    b0r, b1r, b2r = b0.reshape(1, 128), b1.reshape(1, 128), b2.reshape(1, 128)
    rb0r, rb1r = rb0.reshape(1, BGRAPH), rb1.reshape(1, 32)
    rw2b = jnp.broadcast_to(rW2, (32, 128))
    rb2r = jnp.broadcast_to(rb2.reshape(1, 1), (1, 128))

    embw0 = _prep_call(emb_pad, W0)[:100]
    idx = _knn_call(lo_al, nt, pos8, batch3, posT8, batchT8)
    h0 = _lookup_call(embw0, z3)

    idx4 = idx.T[:KNN].reshape(KNN, NW, NCH, CH)

    acc0 = _agg_call(h0, idx4)
    h1 = _fused_call(acc0, h0, b0r, W1)
    acc1 = _agg_call(h1, idx4)
    h2 = _fused_call(acc1, h1, b1r, W2)
    acc2 = _agg_call(h2, idx4)
    res = _final_call(acc2, h2, b2r, batch3, invc,
                      rW0, rb0r, rW1, rb1r, rw2b, rb2r)
    return res[:, 0]
